# baseline (speedup 1.0000x reference)
import jax
import jax.numpy as jnp
from jax.experimental import pallas as pl

_INTERVAL = 4
_GROUPS_PER_BLOCK = 256


def _skip_kernel(x_ref, o_ref):
    j = pl.program_id(1)

    @pl.when(j == 0)
    def _zero():
        o_ref[...] = jnp.zeros_like(o_ref)

    @pl.when(j != 0)
    def _copy():
        o_ref[...] = x_ref[...]


def kernel(x):
    n, d = x.shape
    g = n // _INTERVAL
    xr = x.reshape(g, _INTERVAL, 1, d)
    out = pl.pallas_call(
        _skip_kernel,
        grid=(g // _GROUPS_PER_BLOCK, _INTERVAL),
        in_specs=[pl.BlockSpec(
            (_GROUPS_PER_BLOCK, 1, 1, d),
            lambda i, j: (i, jnp.maximum(j, 1), 0, 0))],
        out_specs=pl.BlockSpec(
            (_GROUPS_PER_BLOCK, 1, 1, d), lambda i, j: (i, j, 0, 0)),
        out_shape=jax.ShapeDtypeStruct((g, _INTERVAL, 1, d), x.dtype),
    )(xr)
    return out.reshape(n, d)


# speedup vs baseline: 2.6511x; 2.6511x over previous
import functools

import jax
import jax.numpy as jnp
from jax import lax
from jax.experimental import pallas as pl
from jax.experimental.pallas import tpu as pltpu
from jax.experimental.pallas import tpu_sc as plsc

_INTERVAL = 4
_D = 2048
_NC = 2
_NS = 16
_NW = _NC * _NS
_CHUNK_G = 8


def _sc_body(x_hbm, o_hbm, vbuf, zbuf, sem):
    wid = lax.axis_index("s") * _NC + lax.axis_index("c")
    g_total = x_hbm.shape[0]
    gpw = g_total // _NW
    g0 = wid * gpw

    def _zero_init(gi, _):
        def _zrow(i, _):
            zbuf[gi, 0, pl.ds(i * 16, 16)] = jnp.zeros((16,), jnp.float32)
            return 0
        return lax.fori_loop(0, _D // 16, _zrow, 0)

    lax.fori_loop(0, _CHUNK_G, _zero_init, 0)

    def _chunk(ci, _):
        g = g0 + ci * _CHUNK_G
        pltpu.async_copy(
            x_hbm.at[pl.ds(g, _CHUNK_G), pl.ds(1, _INTERVAL - 1), :],
            vbuf, sem).wait()
        pltpu.sync_copy(
            vbuf, o_hbm.at[pl.ds(g, _CHUNK_G), pl.ds(1, _INTERVAL - 1), :])
        pltpu.sync_copy(
            zbuf, o_hbm.at[pl.ds(g, _CHUNK_G), pl.ds(0, 1), :])
        return 0

    lax.fori_loop(0, gpw // _CHUNK_G, _chunk, 0)


def kernel(x):
    n, d = x.shape
    g_total = n // _INTERVAL
    mesh = plsc.VectorSubcoreMesh(core_axis_name="c", subcore_axis_name="s")
    kfn = pl.kernel(
        _sc_body,
        mesh=mesh,
        out_type=jax.ShapeDtypeStruct((g_total, _INTERVAL, d), x.dtype),
        scratch_types=[
            pltpu.VMEM((_CHUNK_G, _INTERVAL - 1, _D), jnp.float32),
            pltpu.VMEM((_CHUNK_G, 1, _D), jnp.float32),
            pltpu.SemaphoreType.DMA,
        ],
    )
    out = kfn(x.reshape(g_total, _INTERVAL, d))
    return out.reshape(n, d)
